# baseline (speedup 1.0000x reference)
import functools

import jax
import jax.numpy as jnp
from jax import lax
from jax.experimental import pallas as pl
from jax.experimental.pallas import tpu as pltpu
from jax.experimental.pallas import tpu_sc as plsc

D = 64
ADD = 16

NC = 2
NS = 16
NW = NC * NS
PASS_T = 640


def _sc_gather_body(table_hbm, idx_hbm, out_hbm, idx_v, rows_v, gsem, osem):
    n_flat = idx_hbm.shape[0]
    tok_w = n_flat // NW
    npass = tok_w // PASS_T
    wid = lax.axis_index("s") * NC + lax.axis_index("c")
    base = wid * tok_w
    pltpu.sync_copy(idx_hbm.at[pl.ds(base, tok_w)], idx_v)
    oh = {}
    for h in range(npass):
        buf = rows_v.at[h % 2]
        idx = idx_v.at[pl.ds(h * PASS_T, PASS_T)]
        if h >= 2:
            oh[h - 2].wait()
        gh = []
        for d in range(D):
            gh.append(pltpu.async_copy(table_hbm.at[d].at[idx], buf.at[d],
                                       gsem))
        for g in gh:
            g.wait()
        oh[h] = pltpu.async_copy(
            buf, out_hbm.at[:, pl.ds(base + h * PASS_T, PASS_T)], osem)
    for h in range(max(0, npass - 2), npass):
        oh[h].wait()


def _sc_gather(table_t, idx_flat):
    n_flat = idx_flat.shape[0]
    mesh = plsc.VectorSubcoreMesh(core_axis_name="c", subcore_axis_name="s")
    return pl.kernel(
        _sc_gather_body,
        out_type=jax.ShapeDtypeStruct((D, n_flat), jnp.float32),
        mesh=mesh,
        scratch_types=[
            pltpu.VMEM((n_flat // NW,), jnp.int32),
            pltpu.VMEM((2, D, PASS_T), jnp.float32),
            pltpu.SemaphoreType.DMA,
            pltpu.SemaphoreType.DMA,
        ],
        compiler_params=pltpu.CompilerParams(use_tc_tiling_on_sc=False),
    )(table_t, idx_flat)


def _mm_body(x_ref, pad_ref, w_ref, o_ref):
    w1 = w_ref[0:D, :]
    w2 = w_ref[D:, :]
    acc = lax.dot_general(
        w1, x_ref[...], (((0,), (0,)), ((), ())),
        preferred_element_type=jnp.float32)
    acc += lax.dot_general(
        w2, pad_ref[0], (((0,), (0,)), ((), ())),
        preferred_element_type=jnp.float32)
    o_ref[...] = acc


def _tc_matmul(sym_t, pad_t, w_rev, l_ctx, bn):
    n_flat = sym_t.shape[1]
    b = n_flat // l_ctx
    rv = w_rev.shape[1]
    nb = b // bn
    return pl.pallas_call(
        _mm_body,
        grid=(l_ctx, nb),
        in_specs=[
            pl.BlockSpec((D, bn), lambda l, j: (0, l * nb + j)),
            pl.BlockSpec((1, ADD, bn), lambda l, j: (l, 0, j)),
            pl.BlockSpec((D + ADD, rv), lambda l, j: (0, 0)),
        ],
        out_specs=pl.BlockSpec((rv, bn), lambda l, j: (l, j)),
        out_shape=jax.ShapeDtypeStruct((l_ctx * rv, b), jnp.float32),
        compiler_params=pltpu.CompilerParams(
            dimension_semantics=("parallel", "parallel")),
    )(sym_t, pad_t, w_rev)


def kernel(ids, table, w_rev, padding):
    b, l = ids.shape
    n_flat = b * l
    rv = w_rev.shape[1]
    idx_flat = ids.T.reshape(n_flat)
    sym_t = _sc_gather(table.T, idx_flat)
    pad_t = padding.transpose(1, 2, 0)
    out_t = _tc_matmul(sym_t, pad_t, w_rev, l, bn=4096)
    return out_t.reshape(l, rv, b).transpose(2, 0, 1)


# speedup vs baseline: 6.9150x; 6.9150x over previous
import functools

import jax
import jax.numpy as jnp
from jax import lax
from jax.experimental import pallas as pl
from jax.experimental.pallas import tpu as pltpu
from jax.experimental.pallas import tpu_sc as plsc

D = 64
ADD = 16

NC = 2
NS = 16
NW = NC * NS
CHUNK = 128


def _sc_gather_body(table_hbm, idx_hbm, out_hbm, idx_v, rows_v, gsem, osem):
    l_ctx, b = idx_hbm.shape
    nchunk = l_ctx
    wid = lax.axis_index("s") * NC + lax.axis_index("c")
    col = wid * CHUNK
    pltpu.sync_copy(idx_hbm.at[:, pl.ds(col, CHUNK)], idx_v)
    gh = {}
    oh = {}
    for j in range(nchunk):
        if j >= 4:
            oh[j - 4].wait()
        gh[j] = pltpu.async_copy(table_hbm.at[idx_v.at[j]], rows_v.at[j % 4],
                                 gsem)
        if j >= 1:
            gh[j - 1].wait()
            oh[j - 1] = pltpu.async_copy(
                rows_v.at[(j - 1) % 4],
                out_hbm.at[pl.ds((j - 1) * b + col, CHUNK)], osem)
    gh[nchunk - 1].wait()
    oh[nchunk - 1] = pltpu.async_copy(
        rows_v.at[(nchunk - 1) % 4],
        out_hbm.at[pl.ds((nchunk - 1) * b + col, CHUNK)], osem)
    for j in range(max(0, nchunk - 4), nchunk):
        oh[j].wait()


def _sc_gather(table, ids_t):
    l_ctx, b = ids_t.shape
    n_flat = l_ctx * b
    mesh = plsc.VectorSubcoreMesh(core_axis_name="c", subcore_axis_name="s")
    return pl.kernel(
        _sc_gather_body,
        out_type=jax.ShapeDtypeStruct((n_flat, D), jnp.float32),
        mesh=mesh,
        scratch_types=[
            pltpu.VMEM((l_ctx, CHUNK), jnp.int32),
            pltpu.VMEM((4, CHUNK, D), jnp.float32),
            pltpu.SemaphoreType.DMA,
            pltpu.SemaphoreType.DMA,
        ],
        compiler_params=pltpu.CompilerParams(use_tc_tiling_on_sc=False),
    )(table, ids_t)


def _mm_body(x_ref, pad_ref, w_ref, o_ref):
    w1 = w_ref[0:D, :]
    w2 = w_ref[D:, :]
    acc = lax.dot_general(
        w1, x_ref[...], (((0,), (1,)), ((), ())),
        preferred_element_type=jnp.float32)
    acc += lax.dot_general(
        w2, pad_ref[0], (((0,), (0,)), ((), ())),
        preferred_element_type=jnp.float32)
    o_ref[...] = acc


def _tc_matmul(sym, pad_t, w_rev, l_ctx, bn):
    n_flat = sym.shape[0]
    b = n_flat // l_ctx
    rv = w_rev.shape[1]
    nb = b // bn
    return pl.pallas_call(
        _mm_body,
        grid=(l_ctx, nb),
        in_specs=[
            pl.BlockSpec((bn, D), lambda l, j: (l * nb + j, 0)),
            pl.BlockSpec((1, ADD, bn), lambda l, j: (l, 0, j)),
            pl.BlockSpec((D + ADD, rv), lambda l, j: (0, 0)),
        ],
        out_specs=pl.BlockSpec((rv, bn), lambda l, j: (l, j)),
        out_shape=jax.ShapeDtypeStruct((l_ctx * rv, b), jnp.float32),
        compiler_params=pltpu.CompilerParams(
            dimension_semantics=("parallel", "parallel")),
    )(sym, pad_t, w_rev)


def kernel(ids, table, w_rev, padding):
    b, l = ids.shape
    rv = w_rev.shape[1]
    sym = _sc_gather(table, ids.T)
    pad_t = padding.transpose(1, 2, 0)
    out_t = _tc_matmul(sym, pad_t, w_rev, l, bn=4096)
    return out_t.reshape(l, rv, b).transpose(2, 0, 1)
